# baseline (speedup 1.0000x reference)
import functools

import jax
import jax.numpy as jnp
from jax import lax
from jax.experimental import pallas as pl
from jax.experimental.pallas import tpu as pltpu
from jax.experimental.pallas import tpu_sc as plsc

N_FACTORS = 32
BATCH = 16384
LANES = 16
CHUNK = 128
PACK = 128 // N_FACTORS
UBLK = 2048
QROWS = 262144


def _pack_block(t0_ref, t1_ref, t2_ref, t3_ref, out_ref):
    eye = jnp.eye(N_FACTORS, dtype=jnp.float32)
    for p, ref in enumerate((t0_ref, t1_ref, t2_ref, t3_ref)):
        out_ref[:, p * N_FACTORS:(p + 1) * N_FACTORS] = lax.dot_general(
            ref[...], eye, (((0,), (0,)), ((), ())))


def _pack_table(tbl_t):
    nblk = QROWS // UBLK
    last_blk = (tbl_t.shape[1] - 1) // UBLK
    return pl.pallas_call(
        _pack_block,
        grid=(nblk,),
        in_specs=[
            pl.BlockSpec(
                (N_FACTORS, UBLK),
                lambda g, p=p: (0, jnp.minimum(p * nblk + g, last_blk)))
            for p in range(PACK)
        ],
        out_specs=pl.BlockSpec((UBLK, 128), lambda g: (g, 0)),
        out_shape=jax.ShapeDtypeStruct((QROWS, 128), jnp.float32),
    )(tbl_t, tbl_t, tbl_t, tbl_t)


def _build_sc_kernel():
    info = plsc.get_sparse_core_info()
    nc, ns = info.num_cores, info.num_subcores
    nw = nc * ns
    b_per_w = BATCH // nw
    n_chunks = b_per_w // CHUNK
    groups_per_chunk = CHUNK // LANES

    mesh = plsc.VectorSubcoreMesh(core_axis_name="c", subcore_axis_name="s")

    @functools.partial(
        pl.kernel,
        mesh=mesh,
        compiler_params=pltpu.CompilerParams(needs_layout_passes=False),
        out_type=jax.ShapeDtypeStruct((BATCH,), jnp.float32),
        scratch_types=[
            pltpu.VMEM((n_chunks, CHUNK), jnp.int32),
            pltpu.VMEM((n_chunks, CHUNK), jnp.int32),
            pltpu.VMEM((n_chunks, CHUNK), jnp.int32),
            pltpu.VMEM((n_chunks, CHUNK), jnp.int32),
            pltpu.VMEM((2, CHUNK, 128), jnp.float32),
            pltpu.VMEM((2, CHUNK, 128), jnp.float32),
            pltpu.VMEM((n_chunks, CHUNK), jnp.float32),
            pltpu.VMEM((n_chunks, CHUNK), jnp.float32),
            pltpu.VMEM((LANES,), jnp.float32),
            pltpu.VMEM((b_per_w,), jnp.float32),
            pltpu.SemaphoreType.DMA,
        ],
    )
    def mf_kernel(user_hbm, item_hbm, uf_hbm, if_hbm, ub_hbm, ib_hbm,
                  gm_hbm, out_hbm,
                  idx_u, idx_i, idxq_u, idxq_i, uf_buf, if_buf,
                  ub_rows, ib_rows, gm_v, out_v, sem):
        wid = lax.axis_index("s") * nc + lax.axis_index("c")
        base = wid * b_per_w

        pltpu.sync_copy(gm_hbm, gm_v)
        for j in range(n_chunks):
            pltpu.sync_copy(user_hbm.at[pl.ds(base + j * CHUNK, CHUNK)],
                            idx_u.at[j])
            pltpu.sync_copy(item_hbm.at[pl.ds(base + j * CHUNK, CHUNK)],
                            idx_i.at[j])

        for j in range(n_chunks):
            for go in range(groups_per_chunk):
                sl = pl.ds(go * LANES, LANES)
                idxq_u[j, sl] = idx_u[j, sl] & (QROWS - 1)
                idxq_i[j, sl] = idx_i[j, sl] & (QROWS - 1)

        def fire(j):
            return [
                pltpu.async_copy(uf_hbm.at[idxq_u.at[j]], uf_buf.at[j % 2],
                                 sem),
                pltpu.async_copy(if_hbm.at[idxq_i.at[j]], if_buf.at[j % 2],
                                 sem),
                pltpu.async_copy(ub_hbm.at[idx_u.at[j]], ub_rows.at[j], sem),
                pltpu.async_copy(ib_hbm.at[idx_i.at[j]], ib_rows.at[j], sem),
            ]

        gm = gm_v[...]
        lane = lax.iota(jnp.int32, LANES)

        inflight = fire(0)
        for j in range(n_chunks):
            done, inflight = inflight, (fire(j + 1)
                                        if j + 1 < n_chunks else [])
            for c in done:
                c.wait()
            buf = j % 2

            def body(go, carry, j=j, buf=buf):
                sl = pl.ds(go * LANES, LANES)
                pu = lax.shift_right_logical(idx_u[j, sl], 18) * N_FACTORS
                pi = lax.shift_right_logical(idx_i[j, sl], 18) * N_FACTORS
                rid = go * LANES + lane
                acc = gm + ub_rows[j, sl] + ib_rows[j, sl]
                for f in range(N_FACTORS):
                    acc = acc + (
                        plsc.load_gather(uf_buf.at[buf], [rid, pu + f])
                        * plsc.load_gather(if_buf.at[buf], [rid, pi + f]))
                out_v[pl.ds(j * CHUNK + go * LANES, LANES)] = acc
                return carry

            lax.fori_loop(0, groups_per_chunk, body, 0)

        pltpu.sync_copy(out_v, out_hbm.at[pl.ds(base, b_per_w)])

    return mf_kernel


def kernel(user, item, user_factors, item_factors, user_bias, item_bias,
           global_mean):
    gm16 = jnp.broadcast_to(
        jnp.asarray(global_mean, jnp.float32).reshape(()), (LANES,))
    uf2 = _pack_table(user_factors.T)
    if2 = _pack_table(item_factors.T)
    mf = _build_sc_kernel()
    return mf(user.astype(jnp.int32), item.astype(jnp.int32), uf2, if2,
              user_bias.reshape(-1), item_bias.reshape(-1), gm16)


# speedup vs baseline: 1.0269x; 1.0269x over previous
import functools

import jax
import jax.numpy as jnp
from jax import lax
from jax.experimental import pallas as pl
from jax.experimental.pallas import tpu as pltpu
from jax.experimental.pallas import tpu_sc as plsc

N_FACTORS = 32
BATCH = 16384
LANES = 16
CHUNK = 128
PACK = 128 // N_FACTORS
UBLK = 2048
QROWS = 262144


def _pack_block(*refs):
    eye = jnp.eye(N_FACTORS, dtype=jnp.float32)
    u_refs, i_refs, uo_ref, io_ref = refs[:4], refs[4:8], refs[8], refs[9]
    for out_ref, in_refs in ((uo_ref, u_refs), (io_ref, i_refs)):
        for p, ref in enumerate(in_refs):
            out_ref[:, p * N_FACTORS:(p + 1) * N_FACTORS] = lax.dot_general(
                ref[...], eye, (((0,), (0,)), ((), ())))


def _pack_tables(uf_t, if_t):
    nblk = QROWS // UBLK
    last_blk = (uf_t.shape[1] - 1) // UBLK
    specs = [
        pl.BlockSpec(
            (N_FACTORS, UBLK),
            lambda g, p=p: (0, jnp.minimum(p * nblk + g, last_blk)))
        for p in range(PACK)
    ]
    out = jax.ShapeDtypeStruct((QROWS, 128), jnp.float32)
    return pl.pallas_call(
        _pack_block,
        grid=(nblk,),
        in_specs=specs + specs,
        out_specs=[pl.BlockSpec((UBLK, 128), lambda g: (g, 0))] * 2,
        out_shape=[out, out],
    )(uf_t, uf_t, uf_t, uf_t, if_t, if_t, if_t, if_t)


def _build_sc_kernel():
    info = plsc.get_sparse_core_info()
    nc, ns = info.num_cores, info.num_subcores
    nw = nc * ns
    b_per_w = BATCH // nw
    n_chunks = b_per_w // CHUNK
    groups_per_chunk = CHUNK // LANES

    mesh = plsc.VectorSubcoreMesh(core_axis_name="c", subcore_axis_name="s")

    @functools.partial(
        pl.kernel,
        mesh=mesh,
        compiler_params=pltpu.CompilerParams(needs_layout_passes=False),
        out_type=jax.ShapeDtypeStruct((BATCH,), jnp.float32),
        scratch_types=[
            pltpu.VMEM((n_chunks, CHUNK), jnp.int32),
            pltpu.VMEM((n_chunks, CHUNK), jnp.int32),
            pltpu.VMEM((n_chunks, CHUNK), jnp.int32),
            pltpu.VMEM((n_chunks, CHUNK), jnp.int32),
            pltpu.VMEM((2, CHUNK, 128), jnp.float32),
            pltpu.VMEM((2, CHUNK, 128), jnp.float32),
            pltpu.VMEM((n_chunks, CHUNK), jnp.float32),
            pltpu.VMEM((n_chunks, CHUNK), jnp.float32),
            pltpu.VMEM((LANES,), jnp.float32),
            pltpu.VMEM((b_per_w,), jnp.float32),
            pltpu.SemaphoreType.DMA,
        ],
    )
    def mf_kernel(user_hbm, item_hbm, uf_hbm, if_hbm, ub_hbm, ib_hbm,
                  gm_hbm, out_hbm,
                  idx_u, idx_i, idxq_u, idxq_i, uf_buf, if_buf,
                  ub_rows, ib_rows, gm_v, out_v, sem):
        wid = lax.axis_index("s") * nc + lax.axis_index("c")
        base = wid * b_per_w

        pltpu.sync_copy(gm_hbm, gm_v)
        for j in range(n_chunks):
            pltpu.sync_copy(user_hbm.at[pl.ds(base + j * CHUNK, CHUNK)],
                            idx_u.at[j])
            pltpu.sync_copy(item_hbm.at[pl.ds(base + j * CHUNK, CHUNK)],
                            idx_i.at[j])

        for j in range(n_chunks):
            for go in range(groups_per_chunk):
                sl = pl.ds(go * LANES, LANES)
                idxq_u[j, sl] = idx_u[j, sl] & (QROWS - 1)
                idxq_i[j, sl] = idx_i[j, sl] & (QROWS - 1)

        def fire(j):
            return [
                pltpu.async_copy(uf_hbm.at[idxq_u.at[j]], uf_buf.at[j % 2],
                                 sem),
                pltpu.async_copy(if_hbm.at[idxq_i.at[j]], if_buf.at[j % 2],
                                 sem),
                pltpu.async_copy(ub_hbm.at[idx_u.at[j]], ub_rows.at[j], sem),
                pltpu.async_copy(ib_hbm.at[idx_i.at[j]], ib_rows.at[j], sem),
            ]

        gm = gm_v[...]
        lane = lax.iota(jnp.int32, LANES)

        inflight = fire(0)
        for j in range(n_chunks):
            done, inflight = inflight, (fire(j + 1)
                                        if j + 1 < n_chunks else [])
            for c in done:
                c.wait()
            buf = j % 2

            def body(go, carry, j=j, buf=buf):
                sl = pl.ds(go * LANES, LANES)
                pu = lax.shift_right_logical(idx_u[j, sl], 18) * N_FACTORS
                pi = lax.shift_right_logical(idx_i[j, sl], 18) * N_FACTORS
                rid = go * LANES + lane
                acc = gm + ub_rows[j, sl] + ib_rows[j, sl]
                for f in range(N_FACTORS):
                    acc = acc + (
                        plsc.load_gather(uf_buf.at[buf], [rid, pu + f])
                        * plsc.load_gather(if_buf.at[buf], [rid, pi + f]))
                out_v[pl.ds(j * CHUNK + go * LANES, LANES)] = acc
                return carry

            lax.fori_loop(0, groups_per_chunk, body, 0)

        pltpu.sync_copy(out_v, out_hbm.at[pl.ds(base, b_per_w)])

    return mf_kernel


def kernel(user, item, user_factors, item_factors, user_bias, item_bias,
           global_mean):
    gm16 = jnp.broadcast_to(
        jnp.asarray(global_mean, jnp.float32).reshape(()), (LANES,))
    uf2, if2 = _pack_tables(user_factors.T, item_factors.T)
    mf = _build_sc_kernel()
    return mf(user.astype(jnp.int32), item.astype(jnp.int32), uf2, if2,
              user_bias.reshape(-1), item_bias.reshape(-1), gm16)


# speedup vs baseline: 1.1905x; 1.1593x over previous
import functools

import jax
import jax.numpy as jnp
from jax import lax
from jax.experimental import pallas as pl
from jax.experimental.pallas import tpu as pltpu
from jax.experimental.pallas import tpu_sc as plsc

N_FACTORS = 32
BATCH = 16384
LANES = 16
CHUNK = 128
PACK = 128 // N_FACTORS
UBLK = 2048
QROWS = 262144


def _pack_block(*refs):
    eye = jnp.eye(N_FACTORS, dtype=jnp.float32)
    u_refs, i_refs, uo_ref, io_ref = refs[:4], refs[4:8], refs[8], refs[9]
    for out_ref, in_refs in ((uo_ref, u_refs), (io_ref, i_refs)):
        for p, ref in enumerate(in_refs):
            out_ref[:, p * N_FACTORS:(p + 1) * N_FACTORS] = lax.dot_general(
                ref[...], eye, (((0,), (0,)), ((), ())))


def _pack_tables(uf_t, if_t):
    nblk = QROWS // UBLK
    last_blk = (uf_t.shape[1] - 1) // UBLK
    specs = [
        pl.BlockSpec(
            (N_FACTORS, UBLK),
            lambda g, p=p: (0, jnp.minimum(p * nblk + g, last_blk)))
        for p in range(PACK)
    ]
    out = jax.ShapeDtypeStruct((QROWS, 128), jnp.float32)
    return pl.pallas_call(
        _pack_block,
        grid=(nblk,),
        in_specs=specs + specs,
        out_specs=[pl.BlockSpec((UBLK, 128), lambda g: (g, 0))] * 2,
        out_shape=[out, out],
    )(uf_t, uf_t, uf_t, uf_t, if_t, if_t, if_t, if_t)


def _build_sc_kernel():
    info = plsc.get_sparse_core_info()
    nc, ns = info.num_cores, info.num_subcores
    nw = nc * ns
    b_per_w = BATCH // nw
    n_chunks = b_per_w // CHUNK
    groups_per_chunk = CHUNK // LANES

    mesh = plsc.VectorSubcoreMesh(core_axis_name="c", subcore_axis_name="s")

    @functools.partial(
        pl.kernel,
        mesh=mesh,
        compiler_params=pltpu.CompilerParams(needs_layout_passes=False),
        out_type=jax.ShapeDtypeStruct((BATCH,), jnp.float32),
        scratch_types=[
            pltpu.VMEM((n_chunks, CHUNK), jnp.int32),
            pltpu.VMEM((n_chunks, CHUNK), jnp.int32),
            pltpu.VMEM((n_chunks, CHUNK), jnp.int32),
            pltpu.VMEM((n_chunks, CHUNK), jnp.int32),
            pltpu.VMEM((2, CHUNK, 128), jnp.float32),
            pltpu.VMEM((2, CHUNK, 128), jnp.float32),
            pltpu.VMEM((n_chunks, CHUNK), jnp.float32),
            pltpu.VMEM((n_chunks, CHUNK), jnp.float32),
            pltpu.VMEM((LANES,), jnp.float32),
            pltpu.VMEM((b_per_w,), jnp.float32),
            pltpu.SemaphoreType.DMA,
        ],
    )
    def mf_kernel(user_hbm, item_hbm, uf_hbm, if_hbm, ub_hbm, ib_hbm,
                  gm_hbm, out_hbm,
                  idx_u, idx_i, idxq_u, idxq_i, uf_buf, if_buf,
                  ub_rows, ib_rows, gm_v, out_v, sem):
        wid = lax.axis_index("s") * nc + lax.axis_index("c")
        base = wid * b_per_w

        pltpu.sync_copy(gm_hbm, gm_v)
        for j in range(n_chunks):
            pltpu.sync_copy(user_hbm.at[pl.ds(base + j * CHUNK, CHUNK)],
                            idx_u.at[j])
            pltpu.sync_copy(item_hbm.at[pl.ds(base + j * CHUNK, CHUNK)],
                            idx_i.at[j])

        for j in range(n_chunks):
            for go in range(groups_per_chunk):
                sl = pl.ds(go * LANES, LANES)
                idxq_u[j, sl] = idx_u[j, sl] & (QROWS - 1)
                idxq_i[j, sl] = idx_i[j, sl] & (QROWS - 1)

        def fire(j):
            return [
                pltpu.async_copy(uf_hbm.at[idxq_u.at[j]], uf_buf.at[j % 2],
                                 sem),
                pltpu.async_copy(if_hbm.at[idxq_i.at[j]], if_buf.at[j % 2],
                                 sem),
                pltpu.async_copy(ub_hbm.at[0].at[idx_u.at[j]], ub_rows.at[j],
                                 sem),
                pltpu.async_copy(ib_hbm.at[0].at[idx_i.at[j]], ib_rows.at[j],
                                 sem),
            ]

        gm = gm_v[...]
        lane = lax.iota(jnp.int32, LANES)

        inflight = fire(0)
        for j in range(n_chunks):
            done, inflight = inflight, (fire(j + 1)
                                        if j + 1 < n_chunks else [])
            for c in done:
                c.wait()
            buf = j % 2

            def body(go, carry, j=j, buf=buf):
                sl = pl.ds(go * LANES, LANES)
                pu = lax.shift_right_logical(idx_u[j, sl], 18) * N_FACTORS
                pi = lax.shift_right_logical(idx_i[j, sl], 18) * N_FACTORS
                rid = go * LANES + lane
                acc = gm + ub_rows[j, sl] + ib_rows[j, sl]
                for f in range(N_FACTORS):
                    acc = acc + (
                        plsc.load_gather(uf_buf.at[buf], [rid, pu + f])
                        * plsc.load_gather(if_buf.at[buf], [rid, pi + f]))
                out_v[pl.ds(j * CHUNK + go * LANES, LANES)] = acc
                return carry

            lax.fori_loop(0, groups_per_chunk, body, 0)

        pltpu.sync_copy(out_v, out_hbm.at[pl.ds(base, b_per_w)])

    return mf_kernel


def kernel(user, item, user_factors, item_factors, user_bias, item_bias,
           global_mean):
    gm16 = jnp.broadcast_to(
        jnp.asarray(global_mean, jnp.float32).reshape(()), (LANES,))
    uf2, if2 = _pack_tables(user_factors.T, item_factors.T)
    mf = _build_sc_kernel()
    return mf(user.astype(jnp.int32), item.astype(jnp.int32), uf2, if2,
              user_bias.T, item_bias.T, gm16)


# speedup vs baseline: 1.2189x; 1.0238x over previous
import functools

import jax
import jax.numpy as jnp
from jax import lax
from jax.experimental import pallas as pl
from jax.experimental.pallas import tpu as pltpu
from jax.experimental.pallas import tpu_sc as plsc

N_FACTORS = 32
BATCH = 16384
LANES = 16
CHUNK = 128
PACK = 128 // N_FACTORS
UBLK = 4096
QROWS = 262144


def _pack_block(*refs):
    eye = jnp.eye(N_FACTORS, dtype=jnp.float32)
    u_refs, i_refs, uo_ref, io_ref = refs[:4], refs[4:8], refs[8], refs[9]
    for out_ref, in_refs in ((uo_ref, u_refs), (io_ref, i_refs)):
        for p, ref in enumerate(in_refs):
            out_ref[:, p * N_FACTORS:(p + 1) * N_FACTORS] = lax.dot_general(
                ref[...], eye, (((0,), (0,)), ((), ())))


def _pack_tables(uf_t, if_t):
    nblk = QROWS // UBLK
    last_blk = (uf_t.shape[1] - 1) // UBLK
    specs = [
        pl.BlockSpec(
            (N_FACTORS, UBLK),
            lambda g, p=p: (0, jnp.minimum(p * nblk + g, last_blk)))
        for p in range(PACK)
    ]
    out = jax.ShapeDtypeStruct((QROWS, 128), jnp.float32)
    return pl.pallas_call(
        _pack_block,
        grid=(nblk,),
        in_specs=specs + specs,
        out_specs=[pl.BlockSpec((UBLK, 128), lambda g: (g, 0))] * 2,
        out_shape=[out, out],
    )(uf_t, uf_t, uf_t, uf_t, if_t, if_t, if_t, if_t)


def _build_sc_kernel():
    info = plsc.get_sparse_core_info()
    nc, ns = info.num_cores, info.num_subcores
    nw = nc * ns
    b_per_w = BATCH // nw
    n_chunks = b_per_w // CHUNK
    groups_per_chunk = CHUNK // LANES

    mesh = plsc.VectorSubcoreMesh(core_axis_name="c", subcore_axis_name="s")

    @functools.partial(
        pl.kernel,
        mesh=mesh,
        compiler_params=pltpu.CompilerParams(needs_layout_passes=False),
        out_type=jax.ShapeDtypeStruct((BATCH,), jnp.float32),
        scratch_types=[
            pltpu.VMEM((n_chunks, CHUNK), jnp.int32),
            pltpu.VMEM((n_chunks, CHUNK), jnp.int32),
            pltpu.VMEM((n_chunks, CHUNK), jnp.int32),
            pltpu.VMEM((n_chunks, CHUNK), jnp.int32),
            pltpu.VMEM((2, CHUNK, 128), jnp.float32),
            pltpu.VMEM((2, CHUNK, 128), jnp.float32),
            pltpu.VMEM((n_chunks, CHUNK), jnp.float32),
            pltpu.VMEM((n_chunks, CHUNK), jnp.float32),
            pltpu.VMEM((LANES,), jnp.float32),
            pltpu.VMEM((b_per_w,), jnp.float32),
            pltpu.SemaphoreType.DMA,
        ],
    )
    def mf_kernel(user_hbm, item_hbm, uf_hbm, if_hbm, ub_hbm, ib_hbm,
                  gm_hbm, out_hbm,
                  idx_u, idx_i, idxq_u, idxq_i, uf_buf, if_buf,
                  ub_rows, ib_rows, gm_v, out_v, sem):
        wid = lax.axis_index("s") * nc + lax.axis_index("c")
        base = wid * b_per_w

        pltpu.sync_copy(gm_hbm, gm_v)
        for j in range(n_chunks):
            pltpu.sync_copy(user_hbm.at[pl.ds(base + j * CHUNK, CHUNK)],
                            idx_u.at[j])
            pltpu.sync_copy(item_hbm.at[pl.ds(base + j * CHUNK, CHUNK)],
                            idx_i.at[j])

        for j in range(n_chunks):
            for go in range(groups_per_chunk):
                sl = pl.ds(go * LANES, LANES)
                idxq_u[j, sl] = idx_u[j, sl] & (QROWS - 1)
                idxq_i[j, sl] = idx_i[j, sl] & (QROWS - 1)

        def fire(j):
            return [
                pltpu.async_copy(uf_hbm.at[idxq_u.at[j]], uf_buf.at[j % 2],
                                 sem),
                pltpu.async_copy(if_hbm.at[idxq_i.at[j]], if_buf.at[j % 2],
                                 sem),
                pltpu.async_copy(ub_hbm.at[0].at[idx_u.at[j]], ub_rows.at[j],
                                 sem),
                pltpu.async_copy(ib_hbm.at[0].at[idx_i.at[j]], ib_rows.at[j],
                                 sem),
            ]

        gm = gm_v[...]
        lane = lax.iota(jnp.int32, LANES)

        inflight = fire(0)
        for j in range(n_chunks):
            done, inflight = inflight, (fire(j + 1)
                                        if j + 1 < n_chunks else [])
            for c in done:
                c.wait()
            buf = j % 2

            def body(go, carry, j=j, buf=buf):
                sl = pl.ds(go * LANES, LANES)
                pu = lax.shift_right_logical(idx_u[j, sl], 18) * N_FACTORS
                pi = lax.shift_right_logical(idx_i[j, sl], 18) * N_FACTORS
                rid = go * LANES + lane
                acc = gm + ub_rows[j, sl] + ib_rows[j, sl]
                for f in range(N_FACTORS):
                    acc = acc + (
                        plsc.load_gather(uf_buf.at[buf], [rid, pu + f])
                        * plsc.load_gather(if_buf.at[buf], [rid, pi + f]))
                out_v[pl.ds(j * CHUNK + go * LANES, LANES)] = acc
                return carry

            lax.fori_loop(0, groups_per_chunk, body, 0)

        pltpu.sync_copy(out_v, out_hbm.at[pl.ds(base, b_per_w)])

    return mf_kernel


def kernel(user, item, user_factors, item_factors, user_bias, item_bias,
           global_mean):
    gm16 = jnp.broadcast_to(
        jnp.asarray(global_mean, jnp.float32).reshape(()), (LANES,))
    uf2, if2 = _pack_tables(user_factors.T, item_factors.T)
    mf = _build_sc_kernel()
    return mf(user.astype(jnp.int32), item.astype(jnp.int32), uf2, if2,
              user_bias.T, item_bias.T, gm16)


# speedup vs baseline: 1.2325x; 1.0111x over previous
import functools

import jax
import jax.numpy as jnp
from jax import lax
from jax.experimental import pallas as pl
from jax.experimental.pallas import tpu as pltpu
from jax.experimental.pallas import tpu_sc as plsc

N_FACTORS = 32
BATCH = 16384
LANES = 16
CHUNK = 128
PACK = 128 // N_FACTORS
UBLK = 8192
QROWS = 262144


def _pack_block(*refs):
    eye = jnp.eye(N_FACTORS, dtype=jnp.float32)
    u_refs, i_refs, uo_ref, io_ref = refs[:4], refs[4:8], refs[8], refs[9]
    for out_ref, in_refs in ((uo_ref, u_refs), (io_ref, i_refs)):
        for p, ref in enumerate(in_refs):
            out_ref[:, p * N_FACTORS:(p + 1) * N_FACTORS] = lax.dot_general(
                ref[...], eye, (((0,), (0,)), ((), ())))


def _pack_tables(uf_t, if_t):
    nblk = QROWS // UBLK
    last_blk = (uf_t.shape[1] - 1) // UBLK
    specs = [
        pl.BlockSpec(
            (N_FACTORS, UBLK),
            lambda g, p=p: (0, jnp.minimum(p * nblk + g, last_blk)))
        for p in range(PACK)
    ]
    out = jax.ShapeDtypeStruct((QROWS, 128), jnp.float32)
    return pl.pallas_call(
        _pack_block,
        grid=(nblk,),
        in_specs=specs + specs,
        out_specs=[pl.BlockSpec((UBLK, 128), lambda g: (g, 0))] * 2,
        out_shape=[out, out],
    )(uf_t, uf_t, uf_t, uf_t, if_t, if_t, if_t, if_t)


def _build_sc_kernel():
    info = plsc.get_sparse_core_info()
    nc, ns = info.num_cores, info.num_subcores
    nw = nc * ns
    b_per_w = BATCH // nw
    n_chunks = b_per_w // CHUNK
    groups_per_chunk = CHUNK // LANES

    mesh = plsc.VectorSubcoreMesh(core_axis_name="c", subcore_axis_name="s")

    @functools.partial(
        pl.kernel,
        mesh=mesh,
        compiler_params=pltpu.CompilerParams(needs_layout_passes=False),
        out_type=jax.ShapeDtypeStruct((BATCH,), jnp.float32),
        scratch_types=[
            pltpu.VMEM((n_chunks, CHUNK), jnp.int32),
            pltpu.VMEM((n_chunks, CHUNK), jnp.int32),
            pltpu.VMEM((n_chunks, CHUNK), jnp.int32),
            pltpu.VMEM((n_chunks, CHUNK), jnp.int32),
            pltpu.VMEM((2, CHUNK, 128), jnp.float32),
            pltpu.VMEM((2, CHUNK, 128), jnp.float32),
            pltpu.VMEM((n_chunks, CHUNK), jnp.float32),
            pltpu.VMEM((n_chunks, CHUNK), jnp.float32),
            pltpu.VMEM((LANES,), jnp.float32),
            pltpu.VMEM((b_per_w,), jnp.float32),
            pltpu.SemaphoreType.DMA,
        ],
    )
    def mf_kernel(user_hbm, item_hbm, uf_hbm, if_hbm, ub_hbm, ib_hbm,
                  gm_hbm, out_hbm,
                  idx_u, idx_i, idxq_u, idxq_i, uf_buf, if_buf,
                  ub_rows, ib_rows, gm_v, out_v, sem):
        wid = lax.axis_index("s") * nc + lax.axis_index("c")
        base = wid * b_per_w

        pltpu.sync_copy(gm_hbm, gm_v)
        for j in range(n_chunks):
            pltpu.sync_copy(user_hbm.at[pl.ds(base + j * CHUNK, CHUNK)],
                            idx_u.at[j])
            pltpu.sync_copy(item_hbm.at[pl.ds(base + j * CHUNK, CHUNK)],
                            idx_i.at[j])

        for j in range(n_chunks):
            for go in range(groups_per_chunk):
                sl = pl.ds(go * LANES, LANES)
                idxq_u[j, sl] = idx_u[j, sl] & (QROWS - 1)
                idxq_i[j, sl] = idx_i[j, sl] & (QROWS - 1)

        def fire(j):
            return [
                pltpu.async_copy(uf_hbm.at[idxq_u.at[j]], uf_buf.at[j % 2],
                                 sem),
                pltpu.async_copy(if_hbm.at[idxq_i.at[j]], if_buf.at[j % 2],
                                 sem),
                pltpu.async_copy(ub_hbm.at[0].at[idx_u.at[j]], ub_rows.at[j],
                                 sem),
                pltpu.async_copy(ib_hbm.at[0].at[idx_i.at[j]], ib_rows.at[j],
                                 sem),
            ]

        gm = gm_v[...]
        lane = lax.iota(jnp.int32, LANES)

        inflight = fire(0)
        for j in range(n_chunks):
            done, inflight = inflight, (fire(j + 1)
                                        if j + 1 < n_chunks else [])
            for c in done:
                c.wait()
            buf = j % 2

            def body(go, carry, j=j, buf=buf):
                sl = pl.ds(go * LANES, LANES)
                pu = lax.shift_right_logical(idx_u[j, sl], 18) * N_FACTORS
                pi = lax.shift_right_logical(idx_i[j, sl], 18) * N_FACTORS
                rid = go * LANES + lane
                acc = gm + ub_rows[j, sl] + ib_rows[j, sl]
                for f in range(N_FACTORS):
                    acc = acc + (
                        plsc.load_gather(uf_buf.at[buf], [rid, pu + f])
                        * plsc.load_gather(if_buf.at[buf], [rid, pi + f]))
                out_v[pl.ds(j * CHUNK + go * LANES, LANES)] = acc
                return carry

            lax.fori_loop(0, groups_per_chunk, body, 0)

        pltpu.sync_copy(out_v, out_hbm.at[pl.ds(base, b_per_w)])

    return mf_kernel


def kernel(user, item, user_factors, item_factors, user_bias, item_bias,
           global_mean):
    gm16 = jnp.broadcast_to(
        jnp.asarray(global_mean, jnp.float32).reshape(()), (LANES,))
    uf2, if2 = _pack_tables(user_factors.T, item_factors.T)
    mf = _build_sc_kernel()
    return mf(user.astype(jnp.int32), item.astype(jnp.int32), uf2, if2,
              user_bias.T, item_bias.T, gm16)
